# baseline (speedup 1.0000x reference)
import jax
import jax.numpy as jnp
from jax import lax
from jax.experimental import pallas as pl
from jax.experimental.pallas import tpu as pltpu
from jax.experimental.pallas import tpu_sc as plsc

N = 50000
E = 800000
R = 100
D = 32
NB = 4
BD = NB * D

NC = 2
NS = 16
L = 16
NW = NC * NS

EPAD = 819200
TPAD = 127
KM = 128
NKEY = N * KM
NQ = 4
QS = NKEY // NQ

CH = 128
GPC = CH // L
SCH = 1280
CPS = SCH // CH
EPT_C = EPAD // NW
SUP_C = EPT_C // SCH

CHB = 128
GPB = CHB // L
SCHB = 2560
CPSB = SCHB // CHB
EPT_B = EPAD // NS
SUP_B = EPT_B // SCHB

NPAD = 50048
RPT = NPAD // NS
ZB = 2000

BLK = 128
GRID = NPAD // BLK

_SC_PARAMS = dict(
    compiler_params=pltpu.CompilerParams(
        needs_layout_passes=False, use_tc_tiling_on_sc=False))


def _mesh():
    return plsc.VectorSubcoreMesh(
        core_axis_name="c", subcore_axis_name="s", num_cores=NC,
        num_subcores=NS)


def _keys(dstbuf, typebuf, cc, lo, g):
    o = cc * CHB + g * L
    tv = typebuf[pl.ds(o, L)]
    key = dstbuf[pl.ds(o, L)] * KM + tv
    local = key - lo
    m = (local >= 0) & (local < QS)
    return local, m, tv


def _build_keys(dstbuf, typebuf, kbuf, vbuf, cc, lo):
    for g in range(GPB):
        local, m, _ = _keys(dstbuf, typebuf, cc, lo, g)
        kbuf[pl.ds(g * L, L)] = jnp.clip(local, 0, QS - 1)
        if vbuf is not None:
            vbuf[pl.ds(g * L, L)] = jnp.where(m, 1.0, 0.0).astype(jnp.float32)


def _norm_body(dst_hbm, type_hbm, parts_hbm, counts_sh, zbuf, dstbuf, typebuf,
               k0, k1, v0, v1, c0b, c1b, partbuf, s0, s1):
    c = lax.axis_index("c")
    s = lax.axis_index("s")
    zero = jnp.zeros((L,), jnp.float32)

    def zf(i, carry):
        zbuf[pl.ds(i * L, L)] = zero
        return carry
    lax.fori_loop(0, ZB // L, zf, 0)

    for qi in range(NQ // NC):
        q = c * (NQ // NC) + qi
        lo = q * QS

        def zc(i, carry):
            pltpu.sync_copy(zbuf,
                            counts_sh.at[pl.ds(s * (QS // NS) + i * ZB, ZB)])
            return carry
        lax.fori_loop(0, (QS // NS) // ZB, zc, 0)
        plsc.subcore_barrier()

        def sup1(sp, carry):
            off = s * EPT_B + sp * SCHB
            pltpu.sync_copy(dst_hbm.at[pl.ds(off, SCHB)], dstbuf)
            pltpu.sync_copy(type_hbm.at[pl.ds(off, SCHB)], typebuf)
            _build_keys(dstbuf, typebuf, k0, v0, 0, lo)
            pltpu.async_copy(v0, counts_sh.at[k0], s0, add=True)

            def pair(j, carry2):
                _build_keys(dstbuf, typebuf, k1, v1, 2 * j + 1, lo)
                pltpu.async_copy(v1, counts_sh.at[k1], s1, add=True)
                pltpu.make_async_copy(v0, counts_sh.at[k0], s0).wait()

                @pl.when(j < CPSB // 2 - 1)
                def _():
                    _build_keys(dstbuf, typebuf, k0, v0, 2 * j + 2, lo)
                    pltpu.async_copy(v0, counts_sh.at[k0], s0, add=True)
                pltpu.make_async_copy(v1, counts_sh.at[k1], s1).wait()
                return carry2
            lax.fori_loop(0, CPSB // 2, pair, 0)
            return carry
        lax.fori_loop(0, SUP_B, sup1, 0)
        plsc.subcore_barrier()

        def sup2(sp, carry):
            off = s * EPT_B + sp * SCHB
            pltpu.sync_copy(dst_hbm.at[pl.ds(off, SCHB)], dstbuf)
            pltpu.sync_copy(type_hbm.at[pl.ds(off, SCHB)], typebuf)
            _build_keys(dstbuf, typebuf, k0, None, 0, lo)
            pltpu.async_copy(counts_sh.at[k0], c0b, s0)

            def norms(cc, cbuf):
                for g in range(GPB):
                    local, m, tv = _keys(dstbuf, typebuf, cc, lo, g)
                    m = m & (tv < R)
                    cnt = cbuf[pl.ds(g * L, L)]
                    partbuf[pl.ds(cc * CHB + g * L, L)] = jnp.where(
                        m, 1.0 / cnt, 0.0)

            def pair(j, carry2):
                _build_keys(dstbuf, typebuf, k1, None, 2 * j + 1, lo)
                pltpu.async_copy(counts_sh.at[k1], c1b, s1)
                pltpu.make_async_copy(counts_sh.at[k0], c0b, s0).wait()
                norms(2 * j, c0b)

                @pl.when(j < CPSB // 2 - 1)
                def _():
                    _build_keys(dstbuf, typebuf, k0, None, 2 * j + 2, lo)
                    pltpu.async_copy(counts_sh.at[k0], c0b, s0)
                pltpu.make_async_copy(counts_sh.at[k1], c1b, s1).wait()
                norms(2 * j + 1, c1b)
                return carry2
            lax.fori_loop(0, CPSB // 2, pair, 0)
            pltpu.sync_copy(partbuf, parts_hbm.at[pl.ds(q * EPAD + off, SCHB)])
            return carry
        lax.fori_loop(0, SUP_B, sup2, 0)
        plsc.subcore_barrier()


def _norm_kernel(dstp, typep):
    f = pl.kernel(
        _norm_body,
        out_type=jax.ShapeDtypeStruct((NQ * EPAD,), jnp.float32),
        mesh=_mesh(),
        scratch_types=[
            pltpu.VMEM_SHARED((QS,), jnp.float32),
            pltpu.VMEM((ZB,), jnp.float32),
            pltpu.VMEM((SCHB,), jnp.int32),
            pltpu.VMEM((SCHB,), jnp.int32),
            pltpu.VMEM((CHB,), jnp.int32),
            pltpu.VMEM((CHB,), jnp.int32),
            pltpu.VMEM((CHB,), jnp.float32),
            pltpu.VMEM((CHB,), jnp.float32),
            pltpu.VMEM((CHB,), jnp.float32),
            pltpu.VMEM((CHB,), jnp.float32),
            pltpu.VMEM((SCHB,), jnp.float32),
            pltpu.SemaphoreType.DMA,
            pltpu.SemaphoreType.DMA,
        ],
        **_SC_PARAMS,
    )
    return f(dstp, typep)


def _build_idx(srcbuf, dstbuf, srcidx, dstidx, cc):
    for g in range(GPC):
        o = cc * CH + g * L
        srcidx[pl.ds(g * L, L)] = srcbuf[pl.ds(o, L)]
        dstidx[pl.ds(g * L, L)] = dstbuf[pl.ds(o, L)]


def _chunk_msgs(typebuf, normsum, coeffbuf, zrows, msgs, cc, iota2):
    for g in range(GPC):
        o = cc * CH + g * L
        nv = normsum[pl.ds(o, L)]
        tb = typebuf[pl.ds(o, L)] * NB
        ws = [plsc.load_gather(coeffbuf, [tb + b]) * nv for b in range(NB)]

        def eloop(e2, carry):
            e = g * L + e2
            eidx = jnp.full((L,), e2, jnp.int32)
            efull = jnp.full((L,), e, jnp.int32)
            mA = jnp.zeros((L,), jnp.float32)
            mB = jnp.zeros((L,), jnp.float32)
            for b in range(NB):
                zb = zrows[e, pl.ds(b * D, 2 * L)]
                zA, zB = plsc.unpack(zb, format=plsc.PackFormat.INTERLEAVED)
                wb = lax.gather(
                    ws[b], eidx[:, None],
                    lax.GatherDimensionNumbers(
                        offset_dims=(), collapsed_slice_dims=(0,),
                        start_index_map=(0,)),
                    (1,), mode=lax.GatherScatterMode.PROMISE_IN_BOUNDS)
                mA = mA + zA * wb
                mB = mB + zB * wb
            plsc.store_scatter(msgs, [efull, iota2], mA)
            plsc.store_scatter(msgs, [efull, iota2 + 1], mB)
            return carry
        lax.fori_loop(0, L, eloop, 0, unroll=4)


def _mp_body(src_hbm, dst_hbm, type_hbm, z_hbm, parts_hbm, coeff_hbm,
             out_hbm, acc_sh, coeffbuf, srcbuf, dstbuf, typebuf,
             ptmp, normsum, si0, si1, di0, di1, zr0, zr1, msgs, g0, g1):
    c = lax.axis_index("c")
    s = lax.axis_index("s")
    wid = c * NS + s
    iota2 = lax.iota(jnp.int32, L) * 2
    zero = jnp.zeros((L,), jnp.float32)

    pltpu.sync_copy(coeff_hbm, coeffbuf)
    for r in range(CH):
        msgs[r, pl.ds(0, L)] = zero
        msgs[r, pl.ds(L, L)] = zero

    def za(i, carry):
        pltpu.sync_copy(msgs, acc_sh.at[pl.ds(s * RPT + i * CH, CH)])
        return carry
    lax.fori_loop(0, RPT // CH, za, 0)
    pltpu.sync_copy(msgs.at[pl.ds(0, RPT % CH)],
                    acc_sh.at[pl.ds(s * RPT + (RPT // CH) * CH, RPT % CH)])
    plsc.subcore_barrier()

    def sup(sp, carry):
        off = wid * EPT_C + sp * SCH
        pltpu.sync_copy(src_hbm.at[pl.ds(off, SCH)], srcbuf)
        pltpu.sync_copy(dst_hbm.at[pl.ds(off, SCH)], dstbuf)
        pltpu.sync_copy(type_hbm.at[pl.ds(off, SCH)], typebuf)
        pltpu.sync_copy(parts_hbm.at[pl.ds(off, SCH)], normsum)
        for qq in range(1, NQ):
            pltpu.sync_copy(parts_hbm.at[pl.ds(qq * EPAD + off, SCH)], ptmp)

            def acc_p(i, carry2):
                o = i * L
                normsum[pl.ds(o, L)] = (normsum[pl.ds(o, L)]
                                        + ptmp[pl.ds(o, L)])
                return carry2
            lax.fori_loop(0, SCH // L, acc_p, 0)

        _build_idx(srcbuf, dstbuf, si0, di0, 0)
        pltpu.async_copy(z_hbm.at[si0], zr0, g0)

        def pair(j, carry2):
            _build_idx(srcbuf, dstbuf, si1, di1, 2 * j + 1)
            pltpu.async_copy(z_hbm.at[si1], zr1, g1)
            pltpu.make_async_copy(z_hbm.at[si0], zr0, g0).wait()
            _chunk_msgs(typebuf, normsum, coeffbuf, zr0, msgs, 2 * j, iota2)
            pltpu.sync_copy(msgs, acc_sh.at[di0], add=True)

            @pl.when(j < CPS // 2 - 1)
            def _():
                _build_idx(srcbuf, dstbuf, si0, di0, 2 * j + 2)
                pltpu.async_copy(z_hbm.at[si0], zr0, g0)
            pltpu.make_async_copy(z_hbm.at[si1], zr1, g1).wait()
            _chunk_msgs(typebuf, normsum, coeffbuf, zr1, msgs, 2 * j + 1, iota2)
            pltpu.sync_copy(msgs, acc_sh.at[di1], add=True)
            return carry2
        lax.fori_loop(0, CPS // 2, pair, 0)
        return carry
    lax.fori_loop(0, SUP_C, sup, 0)
    plsc.subcore_barrier()

    def co(i, carry):
        r0 = s * RPT + i * CH
        pltpu.sync_copy(acc_sh.at[pl.ds(r0, CH)],
                        out_hbm.at[c, pl.ds(r0, CH)])
        return carry
    lax.fori_loop(0, RPT // CH, co, 0)
    r0 = s * RPT + (RPT // CH) * CH
    pltpu.sync_copy(acc_sh.at[pl.ds(r0, RPT % CH)],
                    out_hbm.at[c, pl.ds(r0, RPT % CH)])


def _mp_kernel(srcp, dstp, typep, z, parts, coeff_flat):
    f = pl.kernel(
        _mp_body,
        out_type=jax.ShapeDtypeStruct((NC, NPAD, D), jnp.float32),
        mesh=_mesh(),
        scratch_types=[
            pltpu.VMEM_SHARED((NPAD, D), jnp.float32),
            pltpu.VMEM((KM * NB,), jnp.float32),
            pltpu.VMEM((SCH,), jnp.int32),
            pltpu.VMEM((SCH,), jnp.int32),
            pltpu.VMEM((SCH,), jnp.int32),
            pltpu.VMEM((SCH,), jnp.float32),
            pltpu.VMEM((SCH,), jnp.float32),
            pltpu.VMEM((CH,), jnp.int32),
            pltpu.VMEM((CH,), jnp.int32),
            pltpu.VMEM((CH,), jnp.int32),
            pltpu.VMEM((CH,), jnp.int32),
            pltpu.VMEM((CH, BD), jnp.bfloat16),
            pltpu.VMEM((CH, BD), jnp.bfloat16),
            pltpu.VMEM((CH, D), jnp.float32),
            pltpu.SemaphoreType.DMA,
            pltpu.SemaphoreType.DMA,
        ],
        **_SC_PARAMS,
    )
    return f(srcp, dstp, typep, z, parts, coeff_flat)


def _enc_body(x_ref, w_ref, z_ref):
    z_ref[...] = jnp.dot(x_ref[...], w_ref[...],
                         preferred_element_type=jnp.float32
                         ).astype(jnp.bfloat16)


def _mid_body(p_ref, w_ref, z_ref):
    t = jnp.tanh(p_ref[0] + p_ref[1])
    z_ref[...] = jnp.dot(t, w_ref[...], preferred_element_type=jnp.float32
                         ).astype(jnp.bfloat16)


def _fin_body(p_ref, o_ref):
    o_ref[...] = jnp.tanh(p_ref[0] + p_ref[1])


def _enc(x, w):
    return pl.pallas_call(
        _enc_body,
        out_shape=jax.ShapeDtypeStruct((NPAD, BD), jnp.bfloat16),
        grid=(GRID,),
        in_specs=[pl.BlockSpec((BLK, D), lambda i: (i, 0)),
                  pl.BlockSpec((D, BD), lambda i: (0, 0))],
        out_specs=pl.BlockSpec((BLK, BD), lambda i: (i, 0)),
    )(x, w)


def _mid(p, w):
    return pl.pallas_call(
        _mid_body,
        out_shape=jax.ShapeDtypeStruct((NPAD, BD), jnp.bfloat16),
        grid=(GRID,),
        in_specs=[pl.BlockSpec((NC, BLK, D), lambda i: (0, i, 0)),
                  pl.BlockSpec((D, BD), lambda i: (0, 0))],
        out_specs=pl.BlockSpec((BLK, BD), lambda i: (i, 0)),
    )(p, w)


def _fin(p):
    return pl.pallas_call(
        _fin_body,
        out_shape=jax.ShapeDtypeStruct((NPAD, D), jnp.float32),
        grid=(GRID,),
        in_specs=[pl.BlockSpec((NC, BLK, D), lambda i: (0, i, 0))],
        out_specs=pl.BlockSpec((BLK, D), lambda i: (i, 0)),
    )(p)


def kernel(ent_ids, edge_index, edge_type, ent_embeds, coeff1, bases1,
           coeff2, bases2):
    x0 = jnp.take(ent_embeds, ent_ids, axis=0)
    x0 = jnp.concatenate([x0, jnp.zeros((NPAD - N, D), jnp.float32)])
    pad = EPAD - E
    srcp = jnp.concatenate([edge_index[0], jnp.zeros((pad,), jnp.int32)])
    dstp = jnp.concatenate([edge_index[1], jnp.zeros((pad,), jnp.int32)])
    typep = jnp.concatenate([edge_type, jnp.full((pad,), TPAD, jnp.int32)])
    w1 = jnp.transpose(bases1, (1, 0, 2)).reshape(D, BD)
    w2 = jnp.transpose(bases2, (1, 0, 2)).reshape(D, BD)
    c1 = jnp.zeros((KM, NB), jnp.float32).at[:R].set(coeff1).reshape(KM * NB)
    c2 = jnp.zeros((KM, NB), jnp.float32).at[:R].set(coeff2).reshape(KM * NB)

    parts = _norm_kernel(dstp, typep)
    z1 = _enc(x0, w1)
    p1 = _mp_kernel(srcp, dstp, typep, z1, parts, c1)
    z2 = _mid(p1, w2)
    p2 = _mp_kernel(srcp, dstp, typep, z2, parts, c2)
    return _fin(p2)[:N]
